# baseline (speedup 1.0000x reference)
import functools

import jax
import jax.numpy as jnp
from jax.experimental import pallas as pl
from jax.experimental.pallas import tpu as pltpu

_K = 7
_W = 10240
_NP = _W // 128

_NEG_INF = float("-inf")
_F32_BIG = 3.0e8

_SORT8 = [(0, 1), (2, 3), (4, 5), (6, 7),
          (0, 2), (1, 3), (4, 6), (5, 7),
          (1, 2), (5, 6),
          (0, 4), (1, 5), (2, 6), (3, 7),
          (2, 4), (3, 5),
          (1, 2), (3, 4), (5, 6)]

_CLEAN8 = [(0, 4), (1, 5), (2, 6), (3, 7),
           (0, 2), (1, 3), (4, 6), (5, 7),
           (0, 1), (2, 3), (4, 5), (6, 7)]


def _ce(a, b):
    (av, ai), (bv, bi) = a, b
    cm = av >= bv
    return ((jnp.maximum(av, bv), jnp.where(cm, ai, bi)),
            (jnp.minimum(av, bv), jnp.where(cm, bi, ai)))


def _keep(a, b):
    (av, ai), (bv, bi) = a, b
    return (jnp.maximum(av, bv), jnp.where(av >= bv, ai, bi))


def _sort8(planes):
    planes = list(planes)
    for i, j in _SORT8:
        planes[i], planes[j] = _ce(planes[i], planes[j])
    return planes


def _merge_keep8(a, b):
    c = [_keep(a[i], b[7 - i]) for i in range(8)]
    for i, j in _CLEAN8:
        c[i], c[j] = _ce(c[i], c[j])
    return c


def _topk_kernel(q_ref, qt_ref, p_ref, pt_ref, mb_ref, lam_ref, out_ref,
                 run_v_ref, run_i_ref, *, n_steps):
    t = pl.program_id(0)
    q = q_ref[...]
    qn = jnp.sqrt(jnp.sum(q * q, axis=1, keepdims=True))
    q = q / jnp.maximum(qn, 1e-8)

    p = p_ref[...]
    pn = jnp.sqrt(jnp.sum(p * p, axis=1, keepdims=True))
    p = p / jnp.maximum(pn, 1e-8)

    sim = jax.lax.dot_general(
        q, p, (((1,), (1,)), ((), ())), preferred_element_type=jnp.float32)

    qt = qt_ref[...]
    pt = pt_ref[...]
    lam = lam_ref[0, 0]
    tw = jnp.exp(-lam * jnp.abs(qt - pt))
    sim = sim * tw + mb_ref[...]

    nq = sim.shape[0]
    lane = jax.lax.broadcasted_iota(jnp.int32, (nq, 128), 1).astype(jnp.float32)
    base = (t * _W).astype(jnp.float32)

    planes = []
    for j in range(_NP):
        v = sim[:, j * 128:(j + 1) * 128]
        idx = lane + (base + jnp.float32(j * 128))
        planes.append((v, idx))

    groups = [_sort8(planes[g * 8:(g + 1) * 8]) for g in range(_NP // 8)]
    while len(groups) > 1:
        groups = [_merge_keep8(groups[i], groups[i + 1])
                  for i in range(0, len(groups) - 1, 2)] + (
                      [groups[-1]] if len(groups) % 2 else [])
    m = groups[0]

    @pl.when(t == 0)
    def _init():
        run_v_ref[...] = jnp.full(run_v_ref.shape, _NEG_INF, jnp.float32)
        run_i_ref[...] = jnp.zeros(run_i_ref.shape, jnp.float32)

    r = [(run_v_ref[:, k * 128:(k + 1) * 128],
          run_i_ref[:, k * 128:(k + 1) * 128]) for k in range(8)]
    r = _merge_keep8(r, m)

    run_v_ref[...] = jnp.concatenate([x[0] for x in r], axis=1)
    run_i_ref[...] = jnp.concatenate([x[1] for x in r], axis=1)

    @pl.when(t == n_steps - 1)
    def _emit():
        av = jnp.concatenate([x[0] for x in r], axis=1)
        ai = jnp.concatenate([x[1] for x in r], axis=1)
        idxs = []
        for _ in range(_K):
            mx = jnp.max(av, axis=1, keepdims=True)
            wi = jnp.min(jnp.where(av == mx, ai, _F32_BIG),
                         axis=1, keepdims=True)
            idxs.append(wi)
            av = jnp.where(ai == wi, _NEG_INF, av)
        out = jnp.concatenate(idxs + [jnp.zeros((nq, 1), jnp.float32)], axis=1)
        out_ref[...] = out.astype(jnp.int32)


def kernel(query_emb, query_time, pool_emb, pool_time, lambda_decay):
    nq, d = query_emb.shape
    n_pool = pool_emb.shape[0]
    n_steps = -(-n_pool // _W)
    p_pad = n_steps * _W

    pool_p = jnp.pad(pool_emb, ((0, p_pad - n_pool), (0, 0)))
    pt_p = jnp.pad(pool_time, (0, p_pad - n_pool)).reshape(1, p_pad)
    qt = query_time.reshape(nq, 1)
    lam = jnp.reshape(lambda_decay, (1, 1)).astype(jnp.float32)
    mask = jnp.where(jnp.arange(p_pad, dtype=jnp.int32) < n_pool,
                     jnp.float32(0), -jnp.inf).reshape(1, p_pad)

    out = pl.pallas_call(
        functools.partial(_topk_kernel, n_steps=n_steps),
        grid=(n_steps,),
        in_specs=[
            pl.BlockSpec((nq, d), lambda t: (0, 0)),
            pl.BlockSpec((nq, 1), lambda t: (0, 0)),
            pl.BlockSpec((_W, d), lambda t: (t, 0)),
            pl.BlockSpec((1, _W), lambda t: (0, t)),
            pl.BlockSpec((1, _W), lambda t: (0, t)),
            pl.BlockSpec((1, 1), lambda t: (0, 0)),
        ],
        out_specs=pl.BlockSpec((nq, 8), lambda t: (0, 0)),
        out_shape=jax.ShapeDtypeStruct((nq, 8), jnp.int32),
        scratch_shapes=[
            pltpu.VMEM((nq, 8 * 128), jnp.float32),
            pltpu.VMEM((nq, 8 * 128), jnp.float32),
        ],
    )(query_emb, qt, pool_p, pt_p, mask, lam)
    return out[:, :_K]


# speedup vs baseline: 1.2009x; 1.2009x over previous
import functools

import jax
import jax.numpy as jnp
from jax.experimental import pallas as pl
from jax.experimental.pallas import tpu as pltpu

_K = 7
_W = 5120
_NP = _W // 128

_NEG_INF = float("-inf")
_F32_BIG = 3.0e8

_SORT8 = [(0, 1), (2, 3), (4, 5), (6, 7),
          (0, 2), (1, 3), (4, 6), (5, 7),
          (1, 2), (5, 6),
          (0, 4), (1, 5), (2, 6), (3, 7),
          (2, 4), (3, 5),
          (1, 2), (3, 4), (5, 6)]

_CLEAN8 = [(0, 4), (1, 5), (2, 6), (3, 7),
           (0, 2), (1, 3), (4, 6), (5, 7),
           (0, 1), (2, 3), (4, 5), (6, 7)]


def _ce(a, b):
    (av, ai), (bv, bi) = a, b
    cm = av >= bv
    return ((jnp.maximum(av, bv), jnp.where(cm, ai, bi)),
            (jnp.minimum(av, bv), jnp.where(cm, bi, ai)))


def _keep(a, b):
    (av, ai), (bv, bi) = a, b
    return (jnp.maximum(av, bv), jnp.where(av >= bv, ai, bi))


def _sort8(planes):
    planes = list(planes)
    for i, j in _SORT8:
        planes[i], planes[j] = _ce(planes[i], planes[j])
    return planes


def _merge_keep8(a, b):
    c = [_keep(a[i], b[7 - i]) for i in range(8)]
    for i, j in _CLEAN8:
        c[i], c[j] = _ce(c[i], c[j])
    return c


def _topk_kernel(q_ref, qt_ref, p_ref, pt_ref, mb_ref, lam_ref, out_ref,
                 run_v_ref, run_i_ref, *, n_steps):
    t = pl.program_id(0)
    q = q_ref[...]
    qn = jnp.sqrt(jnp.sum(q * q, axis=1, keepdims=True))
    q = q / jnp.maximum(qn, 1e-8)

    p = p_ref[...]
    pn = jnp.sqrt(jnp.sum(p * p, axis=1, keepdims=True))
    p = p / jnp.maximum(pn, 1e-8)

    sim = jax.lax.dot_general(
        q, p, (((1,), (1,)), ((), ())), preferred_element_type=jnp.float32)

    qt = qt_ref[...]
    pt = pt_ref[...]
    lam = lam_ref[0, 0]
    tw = jnp.exp(-lam * jnp.abs(qt - pt))
    sim = sim * tw + mb_ref[...]

    nq = sim.shape[0]
    lane = jax.lax.broadcasted_iota(jnp.int32, (nq, 128), 1).astype(jnp.float32)
    base = (t * _W).astype(jnp.float32)

    planes = []
    for j in range(_NP):
        v = sim[:, j * 128:(j + 1) * 128]
        idx = lane + (base + jnp.float32(j * 128))
        planes.append((v, idx))

    groups = [_sort8(planes[g * 8:(g + 1) * 8]) for g in range(_NP // 8)]
    while len(groups) > 1:
        groups = [_merge_keep8(groups[i], groups[i + 1])
                  for i in range(0, len(groups) - 1, 2)] + (
                      [groups[-1]] if len(groups) % 2 else [])
    m = groups[0]

    @pl.when(t == 0)
    def _init():
        run_v_ref[...] = jnp.full(run_v_ref.shape, _NEG_INF, jnp.float32)
        run_i_ref[...] = jnp.zeros(run_i_ref.shape, jnp.float32)

    r = [(run_v_ref[:, k * 128:(k + 1) * 128],
          run_i_ref[:, k * 128:(k + 1) * 128]) for k in range(8)]
    r = _merge_keep8(r, m)

    run_v_ref[...] = jnp.concatenate([x[0] for x in r], axis=1)
    run_i_ref[...] = jnp.concatenate([x[1] for x in r], axis=1)

    @pl.when(t == n_steps - 1)
    def _emit():
        av = jnp.concatenate([x[0] for x in r], axis=1)
        ai = jnp.concatenate([x[1] for x in r], axis=1)
        idxs = []
        for _ in range(_K):
            mx = jnp.max(av, axis=1, keepdims=True)
            wi = jnp.min(jnp.where(av == mx, ai, _F32_BIG),
                         axis=1, keepdims=True)
            idxs.append(wi)
            av = jnp.where(ai == wi, _NEG_INF, av)
        out = jnp.concatenate(idxs + [jnp.zeros((nq, 1), jnp.float32)], axis=1)
        out_ref[...] = out.astype(jnp.int32)


def kernel(query_emb, query_time, pool_emb, pool_time, lambda_decay):
    nq, d = query_emb.shape
    n_pool = pool_emb.shape[0]
    n_steps = -(-n_pool // _W)
    p_pad = n_steps * _W

    pool_p = jnp.pad(pool_emb, ((0, p_pad - n_pool), (0, 0)))
    pt_p = jnp.pad(pool_time, (0, p_pad - n_pool)).reshape(1, p_pad)
    qt = query_time.reshape(nq, 1)
    lam = jnp.reshape(lambda_decay, (1, 1)).astype(jnp.float32)
    mask = jnp.where(jnp.arange(p_pad, dtype=jnp.int32) < n_pool,
                     jnp.float32(0), -jnp.inf).reshape(1, p_pad)

    out = pl.pallas_call(
        functools.partial(_topk_kernel, n_steps=n_steps),
        grid=(n_steps,),
        in_specs=[
            pl.BlockSpec((nq, d), lambda t: (0, 0)),
            pl.BlockSpec((nq, 1), lambda t: (0, 0)),
            pl.BlockSpec((_W, d), lambda t: (t, 0)),
            pl.BlockSpec((1, _W), lambda t: (0, t)),
            pl.BlockSpec((1, _W), lambda t: (0, t)),
            pl.BlockSpec((1, 1), lambda t: (0, 0)),
        ],
        out_specs=pl.BlockSpec((nq, 8), lambda t: (0, 0)),
        out_shape=jax.ShapeDtypeStruct((nq, 8), jnp.int32),
        scratch_shapes=[
            pltpu.VMEM((nq, 8 * 128), jnp.float32),
            pltpu.VMEM((nq, 8 * 128), jnp.float32),
        ],
    )(query_emb, qt, pool_p, pt_p, mask, lam)
    return out[:, :_K]


# speedup vs baseline: 1.2686x; 1.0564x over previous
import functools

import jax
import jax.numpy as jnp
from jax.experimental import pallas as pl
from jax.experimental.pallas import tpu as pltpu

_K = 7
_W = 5120
_NP = _W // 128

_NEG_INF = float("-inf")
_F32_BIG = 3.0e8

_SORT8 = [(0, 1), (2, 3), (4, 5), (6, 7),
          (0, 2), (1, 3), (4, 6), (5, 7),
          (1, 2), (5, 6),
          (0, 4), (1, 5), (2, 6), (3, 7),
          (2, 4), (3, 5),
          (1, 2), (3, 4), (5, 6)]

_CLEAN8 = [(0, 4), (1, 5), (2, 6), (3, 7),
           (0, 2), (1, 3), (4, 6), (5, 7),
           (0, 1), (2, 3), (4, 5), (6, 7)]


def _ce(a, b):
    (av, ai), (bv, bi) = a, b
    cm = av >= bv
    return ((jnp.maximum(av, bv), jnp.where(cm, ai, bi)),
            (jnp.minimum(av, bv), jnp.where(cm, bi, ai)))


def _keep(a, b):
    (av, ai), (bv, bi) = a, b
    return (jnp.maximum(av, bv), jnp.where(av >= bv, ai, bi))


def _sort8(planes):
    planes = list(planes)
    for i, j in _SORT8:
        planes[i], planes[j] = _ce(planes[i], planes[j])
    return planes


def _merge_keep8(a, b):
    c = [_keep(a[i], b[7 - i]) for i in range(8)]
    for i, j in _CLEAN8:
        c[i], c[j] = _ce(c[i], c[j])
    return c


def _topk_kernel(q_ref, qt_ref, p_ref, pt_ref, mb_ref, lam_ref, out_ref,
                 run_ref, *, n_steps):
    t = pl.program_id(0)
    q = q_ref[...]
    qn = jnp.sqrt(jnp.sum(q * q, axis=1, keepdims=True))
    q = q / jnp.maximum(qn, 1e-8)

    p = p_ref[...]
    pn = jnp.sqrt(jnp.sum(p * p, axis=1, keepdims=True))
    p = p / jnp.maximum(pn, 1e-8)

    sim = jax.lax.dot_general(
        q, p, (((1,), (1,)), ((), ())), preferred_element_type=jnp.float32)

    qt = qt_ref[...]
    pt = pt_ref[...]
    lam = lam_ref[0, 0]
    tw = jnp.exp(-lam * jnp.abs(qt - pt))
    sim = sim * tw + mb_ref[...]

    nq = sim.shape[0]
    lane = jax.lax.broadcasted_iota(jnp.int32, (nq, 128), 1).astype(jnp.float32)
    base = (t * _W).astype(jnp.float32)

    planes = []
    for j in range(_NP):
        v = sim[:, j * 128:(j + 1) * 128]
        idx = lane + (base + jnp.float32(j * 128))
        planes.append((v, idx))

    groups = [_sort8(planes[g * 8:(g + 1) * 8]) for g in range(_NP // 8)]
    while len(groups) > 1:
        groups = [_merge_keep8(groups[i], groups[i + 1])
                  for i in range(0, len(groups) - 1, 2)] + (
                      [groups[-1]] if len(groups) % 2 else [])
    m = groups[0]

    @pl.when(t == 0)
    def _init():
        run_ref[...] = jnp.concatenate(
            [jnp.full((nq, 8 * 128), _NEG_INF, jnp.float32),
             jnp.zeros((nq, 8 * 128), jnp.float32)], axis=1)

    r = [(run_ref[:, k * 128:(k + 1) * 128],
          run_ref[:, (8 + k) * 128:(9 + k) * 128]) for k in range(8)]
    r = _merge_keep8(r, m)

    run_ref[...] = jnp.concatenate(
        [x[0] for x in r] + [x[1] for x in r], axis=1)

    @pl.when(t == n_steps - 1)
    def _emit():
        av = jnp.concatenate([x[0] for x in r], axis=1)
        ai = jnp.concatenate([x[1] for x in r], axis=1)
        idxs = []
        for _ in range(_K):
            mx = jnp.max(av, axis=1, keepdims=True)
            wi = jnp.min(jnp.where(av == mx, ai, _F32_BIG),
                         axis=1, keepdims=True)
            idxs.append(wi)
            av = jnp.where(ai == wi, _NEG_INF, av)
        out = jnp.concatenate(idxs + [jnp.zeros((nq, 1), jnp.float32)], axis=1)
        out_ref[...] = out.astype(jnp.int32)


def kernel(query_emb, query_time, pool_emb, pool_time, lambda_decay):
    nq, d = query_emb.shape
    n_pool = pool_emb.shape[0]
    n_steps = -(-n_pool // _W)
    p_pad = n_steps * _W

    pool_p = jnp.pad(pool_emb, ((0, p_pad - n_pool), (0, 0)))
    pt_p = jnp.pad(pool_time, (0, p_pad - n_pool)).reshape(1, p_pad)
    qt = query_time.reshape(nq, 1)
    lam = jnp.reshape(lambda_decay, (1, 1)).astype(jnp.float32)
    mask = jnp.where(jnp.arange(p_pad, dtype=jnp.int32) < n_pool,
                     jnp.float32(0), -jnp.inf).reshape(1, p_pad)

    out = pl.pallas_call(
        functools.partial(_topk_kernel, n_steps=n_steps),
        grid=(n_steps,),
        in_specs=[
            pl.BlockSpec((nq, d), lambda t: (0, 0)),
            pl.BlockSpec((nq, 1), lambda t: (0, 0)),
            pl.BlockSpec((_W, d), lambda t: (t, 0)),
            pl.BlockSpec((1, _W), lambda t: (0, t)),
            pl.BlockSpec((1, _W), lambda t: (0, t)),
            pl.BlockSpec((1, 1), lambda t: (0, 0)),
        ],
        out_specs=pl.BlockSpec((nq, 8), lambda t: (0, 0)),
        out_shape=jax.ShapeDtypeStruct((nq, 8), jnp.int32),
        scratch_shapes=[
            pltpu.VMEM((nq, 16 * 128), jnp.float32),
        ],
    )(query_emb, qt, pool_p, pt_p, mask, lam)
    return out[:, :_K]
